# baseline (speedup 1.0000x reference)
import functools

import jax
import jax.numpy as jnp
from jax import lax
from jax.experimental import pallas as pl
from jax.experimental.pallas import tpu as pltpu
import jax.experimental.pallas.tpu_sc as plsc

_NC, _NS = 2, 16
_NW = _NC * _NS
_CH = 128


def _sc_gather_body(gidx_hbm, table_hbm, out_hbm, idx_v, big_v, sem):
    rows = out_hbm.shape[0] // _NW
    nch = rows * 128 // _CH
    wid = lax.axis_index("s") * _NC + lax.axis_index("c")
    pltpu.sync_copy(gidx_hbm.at[pl.ds(wid * rows * 128 // _CH, nch), :], idx_v)
    fire = 16
    for g in range(nch // fire):
        descs = [
            pltpu.async_copy(
                table_hbm.at[idx_v.at[g * fire + c]],
                big_v.at[(g * fire + c) * _CH // 128],
                sem)
            for c in range(fire)
        ]
        for d in descs:
            d.wait()
    pltpu.sync_copy(big_v, out_hbm.at[pl.ds(wid * rows, rows), :])


def _tanh(x):
    x = jnp.clip(x, -7.99881172180175781, 7.99881172180175781)
    x2 = x * x
    alpha = x * (4.89352455891786e-03 + x2 * (
        6.37261928875436e-04 + x2 * (1.48572235717979e-05 + x2 * (
            5.12229709037114e-08 + x2 * (-8.60467152213735e-11 + x2 * (
                2.00018790482477e-13 + x2 * -2.76076847742355e-16))))))
    beta = 4.89352518554385e-03 + x2 * (2.26843463243900e-03 + x2 * (
        1.18534705686654e-04 + x2 * 1.19825839466702e-06))
    return alpha / beta


def _sigmoid(x):
    return 0.5 * _tanh(0.5 * x) + 0.5


def _gru_body(codes_ref, whh_ref, bhh_ref, p_ref, out_ref, h_ref, hsum_ref):
    t = pl.program_id(0)
    nt = pl.num_programs(0)

    @pl.when(t == 0)
    def _init():
        h_ref[...] = jnp.zeros_like(h_ref)
        hsum_ref[...] = jnp.zeros_like(hsum_ref)

    h = h_ref[...]
    c = codes_ref[0].astype(jnp.float32)
    m1 = c == 1.0
    m2 = c == 2.0
    gh = jnp.dot(h.astype(jnp.bfloat16), whh_ref[...],
                 preferred_element_type=jnp.float32)
    gh = gh + bhh_ref[...]

    def gi(g):
        s = slice(128 * g, 128 * (g + 1))
        return jnp.where(m1, p_ref[1:2, s],
                         jnp.where(m2, p_ref[2:3, s], p_ref[0:1, s]))

    r = _sigmoid(gi(0) + gh[:, 0:128])
    z = _sigmoid(gi(1) + gh[:, 128:256])
    n = _tanh(gi(2) + r * gh[:, 256:384])
    h_new = (1.0 - z) * n + z * h
    h_ref[...] = h_new
    hsum_ref[...] = hsum_ref[...] + h_new

    @pl.when(t == nt - 1)
    def _fin():
        out_ref[...] = hsum_ref[...]


def _head_body(uemb_ref, hsum_ref, vu_ref, vh_ref, red_ref, fcb_ref, out_ref):
    def b16(x):
        return x.astype(jnp.bfloat16).astype(jnp.float32)

    mean = hsum_ref[...] / 50.0
    s = b16(uemb_ref[...]) * vu_ref[...] + b16(mean) * vh_ref[...]
    out_ref[...] = (
        jnp.dot(s, red_ref[...], preferred_element_type=jnp.float32,
                precision=lax.Precision.HIGHEST)
        + fcb_ref[...]
    )


def _pack_rows(x):
    return jnp.tile(x, (1, 8)).reshape(384)


def kernel(user_id, launch_seq, user_table, launch_table, W_ih, W_hh,
           b_ih, b_hh, fc_W, fc_b):
    B, L = launch_seq.shape
    H = W_hh.shape[1]
    M = B // 8

    mesh = plsc.VectorSubcoreMesh(core_axis_name="c", subcore_axis_name="s",
                                  num_cores=_NC, num_subcores=_NS)
    gidx = (user_id.astype(jnp.int32)[:, None] * H
            + jnp.arange(H, dtype=jnp.int32)).reshape(M, 128)
    rows_per_tec = M // _NW
    user_emb = pl.kernel(
        _sc_gather_body,
        out_type=jax.ShapeDtypeStruct((M, 128), jnp.float32),
        mesh=mesh,
        scratch_types=[
            pltpu.VMEM((rows_per_tec, 128), jnp.int32),
            pltpu.VMEM((rows_per_tec, 128), jnp.float32),
            pltpu.SemaphoreType.DMA,
        ],
        compiler_params=pltpu.CompilerParams(use_tc_tiling_on_sc=False),
    )(gidx, user_table.reshape(-1))

    Wg = W_hh.reshape(3, H, H)
    whh_blk = jnp.concatenate(
        [jnp.kron(jnp.eye(8, dtype=W_hh.dtype), Wg[g].T) for g in range(3)],
        axis=1).astype(jnp.bfloat16)
    bhh_row = _pack_rows(b_hh.reshape(3, H)).reshape(1, 384)

    gi_full = jnp.dot(launch_table.astype(jnp.bfloat16),
                      W_ih.T.astype(jnp.bfloat16),
                      preferred_element_type=jnp.float32) + b_ih
    P = jnp.stack([_pack_rows(gi_full[g].reshape(3, H))
                   for g in range(3)])

    codes = launch_seq.astype(jnp.int8).T.reshape(L, M, 8, 1)
    codes = jnp.broadcast_to(codes, (L, M, 8, H)).reshape(L, M, 128)

    hsum = pl.pallas_call(
        _gru_body,
        grid=(L,),
        in_specs=[
            pl.BlockSpec((1, M, 128), lambda t: (t, 0, 0)),
            pl.BlockSpec((128, 384), lambda t: (0, 0)),
            pl.BlockSpec((1, 384), lambda t: (0, 0)),
            pl.BlockSpec((3, 384), lambda t: (0, 0)),
        ],
        out_specs=pl.BlockSpec((M, 128), lambda t: (0, 0)),
        out_shape=jax.ShapeDtypeStruct((M, 128), jnp.float32),
        scratch_shapes=[
            pltpu.VMEM((M, 128), jnp.float32),
            pltpu.VMEM((M, 128), jnp.float32),
        ],
    )(codes, whh_blk, bhh_row, P)

    fcw16 = fc_W.astype(jnp.bfloat16).astype(jnp.float32)
    vu = jnp.tile(fcw16[0, :H], 8).reshape(1, 128)
    vh = jnp.tile(fcw16[0, H:], 8).reshape(1, 128)
    red = jnp.kron(jnp.eye(8, dtype=jnp.float32),
                   jnp.ones((H, 1), dtype=jnp.float32))
    fcb = jnp.broadcast_to(fc_b.reshape(1, 1), (1, 8))

    out = pl.pallas_call(
        _head_body,
        in_specs=[pl.BlockSpec(x.shape, lambda: (0,) * x.ndim)
                  for x in (user_emb, hsum, vu, vh, red, fcb)],
        out_specs=pl.BlockSpec((M, 8), lambda: (0, 0)),
        out_shape=jax.ShapeDtypeStruct((M, 8), jnp.float32),
    )(user_emb, hsum, vu, vh, red, fcb)

    return out.reshape(B, 1)
